# Initial kernel scaffold; baseline (speedup 1.0000x reference)
import jax
import jax.numpy as jnp
from jax.experimental import pallas as pl


def kernel(points, features, W1, g1, b1, W2, g2, b2, W3, g3, b3):
    raise NotImplementedError("write your pallas kernel here")


# speedup vs baseline: 7.5621x; 7.5621x over previous
import functools

import jax
import jax.numpy as jnp
from jax import lax
from jax.experimental import pallas as pl
from jax.experimental.pallas import tpu as pltpu
from jax.experimental.pallas import tpu_sc as plsc

B = 8
N = 4096
S = 512
K = 32
EPS = 1e-5


def _fps_body(px_ref, py_ref, pz_ref, cx_ref, cy_ref, cz_ref):
    px = px_ref[...]
    py = py_ref[...]
    pz = pz_ref[...]
    lane_n = lax.broadcasted_iota(jnp.int32, (B, N), 1)
    lane_s = lax.broadcasted_iota(jnp.int32, (B, S), 1)
    b2 = (px * px + py * py) + pz * pz

    def r16(v):
        return v.astype(jnp.bfloat16).astype(jnp.float32)

    px16 = r16(px)
    py16 = r16(py)
    pz16 = r16(pz)

    cx0 = px[:, 0:1]
    cy0 = py[:, 0:1]
    cz0 = pz[:, 0:1]
    a2 = (cx0 * cx0 + cy0 * cy0) + cz0 * cz0
    cross = (px16 * r16(cx0) + py16 * r16(cy0)) + pz16 * r16(cz0)
    d = (a2 + b2) - 2.0 * cross

    zero_s = jnp.zeros((B, S), jnp.float32)
    sel0 = lane_s == 0
    accx = jnp.where(sel0, cx0, zero_s)
    accy = jnp.where(sel0, cy0, zero_s)
    accz = jnp.where(sel0, cz0, zero_s)

    def body(c, carry):
        d, accx, accy, accz = carry
        m = jnp.max(d, axis=1, keepdims=True)
        ni = jnp.min(jnp.where(d == m, lane_n, N), axis=1, keepdims=True)
        onehot = (lane_n == ni).astype(jnp.float32)
        nx = jnp.sum(px * onehot, axis=1, keepdims=True)
        ny = jnp.sum(py * onehot, axis=1, keepdims=True)
        nz = jnp.sum(pz * onehot, axis=1, keepdims=True)
        selc = lane_s == c
        accx = jnp.where(selc, nx, accx)
        accy = jnp.where(selc, ny, accy)
        accz = jnp.where(selc, nz, accz)
        a2n = (nx * nx + ny * ny) + nz * nz
        crossn = (px16 * r16(nx) + py16 * r16(ny)) + pz16 * r16(nz)
        nd = (a2n + b2) - 2.0 * crossn
        d = jnp.minimum(d, nd)
        return d, accx, accy, accz

    _, accx, accy, accz = lax.fori_loop(1, S, body, (d, accx, accy, accz))
    cx_ref[...] = accx
    cy_ref[...] = accy
    cz_ref[...] = accz


def _fps(px, py, pz, interpret=False):
    out = jax.ShapeDtypeStruct((B, S), jnp.float32)
    return pl.pallas_call(
        _fps_body,
        out_shape=(out, out, out),
        interpret=interpret,
    )(px, py, pz)


_TS = 128


def _knn_body(p_ref, c_ref, kidx_ref):
    P = p_ref[0]
    C = c_ref[0]
    px = P[:, 0:1]
    py = P[:, 1:2]
    pz = P[:, 2:3]
    b2 = (px * px + py * py) + pz * pz
    cx = C[0:1, :]
    cy = C[1:2, :]
    cz = C[2:3, :]
    a2 = (cx * cx + cy * cy) + cz * cz
    cross = lax.dot_general(P, C, (((1,), (0,)), ((), ())),
                            preferred_element_type=jnp.float32)
    d = (a2 + b2) - 2.0 * cross

    iota0 = lax.broadcasted_iota(jnp.int32, (N, _TS), 0)

    def rnd(k, d):
        m = jnp.min(d, axis=0, keepdims=True)
        ni = jnp.min(jnp.where(d == m, iota0, N), axis=0,
                     keepdims=True)
        kidx_ref[0, 0, pl.ds(k, 1), :] = ni
        return jnp.where(iota0 == ni, jnp.inf, d)

    lax.fori_loop(0, K, rnd, d)


def _knn(points, cT, interpret=False):
    grid = (B, S // _TS)
    return pl.pallas_call(
        _knn_body,
        grid=grid,
        in_specs=[
            pl.BlockSpec((1, N, 3), lambda b, t: (b, 0, 0)),
            pl.BlockSpec((1, 3, _TS), lambda b, t: (b, 0, t)),
        ],
        out_specs=pl.BlockSpec((1, 1, K, _TS), lambda b, t: (b, t, 0, 0)),
        out_shape=jax.ShapeDtypeStruct((B, S // _TS, K, _TS), jnp.int32),
        interpret=interpret,
    )(points, cT)


_M = B * S * K
_CH = 128


def _sc_gather(table, idx):
    info = plsc.get_sparse_core_info()
    nw = info.num_cores * info.num_subcores
    rows_per_w = _M // nw
    n_ch = rows_per_w // _CH
    mesh = plsc.VectorSubcoreMesh(core_axis_name="c", subcore_axis_name="s")

    @functools.partial(
        pl.kernel,
        mesh=mesh,
        out_type=jax.ShapeDtypeStruct((_M, 128), jnp.float32),
        scratch_types=[
            pltpu.VMEM((_CH,), jnp.int32),
            pltpu.VMEM((_CH, 128), jnp.float32),
            pltpu.SemaphoreType.DMA,
        ],
    )
    def k(table_hbm, idx_hbm, out_hbm, idx_v, rows_v, sem):
        wid = lax.axis_index("s") * info.num_cores + lax.axis_index("c")
        base = wid * rows_per_w

        def body(i, _):
            off = base + i * _CH
            pltpu.sync_copy(idx_hbm.at[pl.ds(off, _CH)], idx_v)
            pltpu.async_copy(table_hbm.at[idx_v], rows_v, sem).wait()
            pltpu.sync_copy(rows_v, out_hbm.at[pl.ds(off, _CH)])
            return 0

        lax.fori_loop(0, n_ch, body, 0)

    return k(table, idx)


_TM = 2048
_G = _TM // K


def _rowsum8(z):
    return jnp.sum(z.reshape(_TM // 8, 8, z.shape[1]), axis=0)


def _sub_center(z, t):
    c = z.shape[1]
    z3 = z.reshape(_G, K, c)
    return (z3 - t[:, None, :]).reshape(_TM, c)


def _p1_body(x_ref, cp_ref, w1_ref, s_ref, q_ref):
    @pl.when(pl.program_id(0) == 0)
    def _init():
        s_ref[...] = jnp.zeros_like(s_ref)
        q_ref[...] = jnp.zeros_like(q_ref)

    x = x_ref[...][:, :64]
    cp = cp_ref[...]
    w1 = w1_ref[...]
    z = lax.dot_general(x, w1, (((1,), (1,)), ((), ())),
                        preferred_element_type=jnp.float32)
    t = lax.dot_general(cp, w1, (((1,), (1,)), ((), ())),
                        preferred_element_type=jnp.float32)
    z = _sub_center(z, t)
    s_ref[...] += _rowsum8(z)
    q_ref[...] += _rowsum8(z * z)


def _layer1(x, cp, w1f, b1f):
    z = lax.dot_general(x, w1f, (((1,), (1,)), ((), ())),
                        preferred_element_type=jnp.float32)
    t = lax.dot_general(cp, w1f, (((1,), (1,)), ((), ())),
                        preferred_element_type=jnp.float32)
    return jnp.maximum(_sub_center(z, t) + b1f, 0.0)


def _p2_body(x_ref, cp_ref, w1f_ref, b1f_ref, w2_ref, s_ref, q_ref):
    @pl.when(pl.program_id(0) == 0)
    def _init():
        s_ref[...] = jnp.zeros_like(s_ref)
        q_ref[...] = jnp.zeros_like(q_ref)

    a1 = _layer1(x_ref[...][:, :64], cp_ref[...], w1f_ref[...], b1f_ref[...])
    z2 = lax.dot_general(a1, w2_ref[...], (((1,), (1,)), ((), ())),
                         preferred_element_type=jnp.float32)
    s_ref[...] += _rowsum8(z2)
    q_ref[...] += _rowsum8(z2 * z2)


def _p3_body(x_ref, cp_ref, w1f_ref, b1f_ref, w2f_ref, b2f_ref,
             sm_ref, m_ref):
    @pl.when(pl.program_id(0) == 0)
    def _init():
        sm_ref[...] = jnp.zeros_like(sm_ref)
        m_ref[...] = jnp.zeros_like(m_ref)

    a1 = _layer1(x_ref[...][:, :64], cp_ref[...], w1f_ref[...], b1f_ref[...])
    z2 = lax.dot_general(a1, w2f_ref[...], (((1,), (1,)), ((), ())),
                         preferred_element_type=jnp.float32)
    a2 = jnp.maximum(z2 + b2f_ref[...], 0.0)
    sm_ref[...] += lax.dot_general(a2, a2, (((0,), (0,)), ((), ())),
                                   preferred_element_type=jnp.float32)
    m_ref[...] += _rowsum8(a2)


def _p4_body(x_ref, cp_ref, w1f_ref, b1f_ref, w2f_ref, b2f_ref,
             w3f_ref, b3f_ref, o_ref):
    a1 = _layer1(x_ref[...][:, :64], cp_ref[...], w1f_ref[...], b1f_ref[...])
    z2 = lax.dot_general(a1, w2f_ref[...], (((1,), (1,)), ((), ())),
                         preferred_element_type=jnp.float32)
    a2 = jnp.maximum(z2 + b2f_ref[...], 0.0)
    z3 = lax.dot_general(a2, w3f_ref[...], (((1,), (1,)), ((), ())),
                         preferred_element_type=jnp.float32)
    a3 = jnp.maximum(z3 + b3f_ref[...], 0.0)
    o_ref[...] = jnp.max(a3.reshape(_G, K, 256), axis=1)


def _mlp(x0, cpad, W1, g1, b1, W2, g2, b2, W3, g3, b3, interpret=False):
    grid = (_M // _TM,)
    row_spec = lambda c: pl.BlockSpec((_TM, c), lambda i: (i, 0))
    x_spec = pl.BlockSpec((_TM, 128), lambda i: (i, 0))
    cp_spec = pl.BlockSpec((_G, 64), lambda i: (i, 0))
    full = lambda a, b: pl.BlockSpec((a, b), lambda i: (0, 0))

    s1, q1 = pl.pallas_call(
        _p1_body, grid=grid,
        in_specs=[x_spec, cp_spec, full(64, 64)],
        out_specs=(full(8, 64), full(8, 64)),
        out_shape=(jax.ShapeDtypeStruct((8, 64), jnp.float32),) * 2,
        interpret=interpret,
    )(x0, cpad, W1)
    mu1 = jnp.sum(s1, axis=0) / _M
    var1 = jnp.sum(q1, axis=0) / _M - mu1 * mu1
    sc1 = g1 * lax.rsqrt(var1 + EPS)
    w1f = W1 * sc1[:, None]
    b1f = (b1 - sc1 * mu1)[None, :]

    s2, q2 = pl.pallas_call(
        _p2_body, grid=grid,
        in_specs=[x_spec, cp_spec, full(64, 64), full(1, 64),
                  full(128, 64)],
        out_specs=(full(8, 128), full(8, 128)),
        out_shape=(jax.ShapeDtypeStruct((8, 128), jnp.float32),) * 2,
        interpret=interpret,
    )(x0, cpad, w1f, b1f, W2)
    mu2 = jnp.sum(s2, axis=0) / _M
    var2 = jnp.sum(q2, axis=0) / _M - mu2 * mu2
    sc2 = g2 * lax.rsqrt(var2 + EPS)
    w2f = W2 * sc2[:, None]
    b2f = (b2 - sc2 * mu2)[None, :]

    sm, m2 = pl.pallas_call(
        _p3_body, grid=grid,
        in_specs=[x_spec, cp_spec, full(64, 64), full(1, 64),
                  full(128, 64), full(1, 128)],
        out_specs=(full(128, 128), full(8, 128)),
        out_shape=(jax.ShapeDtypeStruct((128, 128), jnp.float32),
                   jax.ShapeDtypeStruct((8, 128), jnp.float32)),
        interpret=interpret,
    )(x0, cpad, w1f, b1f, w2f, b2f)
    mu_a2 = jnp.sum(m2, axis=0) / _M
    mu3 = W3 @ mu_a2
    e3 = jnp.sum((W3 @ (sm / _M)) * W3, axis=1)
    var3 = e3 - mu3 * mu3
    sc3 = g3 * lax.rsqrt(var3 + EPS)
    w3f = W3 * sc3[:, None]
    b3f = (b3 - sc3 * mu3)[None, :]

    out = pl.pallas_call(
        _p4_body, grid=grid,
        in_specs=[x_spec, cp_spec, full(64, 64), full(1, 64),
                  full(128, 64), full(1, 128), full(256, 128), full(1, 256)],
        out_specs=pl.BlockSpec((_G, 256), lambda i: (i, 0)),
        out_shape=jax.ShapeDtypeStruct((_M // K, 256), jnp.float32),
        interpret=interpret,
    )(x0, cpad, w1f, b1f, w2f, b2f, w3f, b3f)
    return out


def kernel(points, features, W1, g1, b1, W2, g2, b2, W3, g3, b3):
    px = points[:, :, 0]
    py = points[:, :, 1]
    pz = points[:, :, 2]
    cx, cy, cz = _fps(px, py, pz)
    cxyz = jnp.stack([cx, cy, cz], axis=-1)
    cT = jnp.stack([cx, cy, cz], axis=1)

    kidx4 = _knn(points, cT)
    kidx = jnp.transpose(kidx4, (0, 1, 3, 2)).reshape(B, S, K)

    flat_idx = (kidx + (jnp.arange(B, dtype=jnp.int32) * N)[:, None, None])
    flat_idx = flat_idx.reshape(_M)
    table = jnp.concatenate([points, features], axis=-1).reshape(B * N, 64)
    table = jnp.pad(table, ((0, 0), (0, 64)))
    x0 = _sc_gather(table, flat_idx)

    cpad = jnp.pad(cxyz.reshape(B * S, 3), ((0, 0), (0, 61)))
    feat = _mlp(x0, cpad, W1, g1, b1, W2, g2, b2, W3, g3, b3)
    return cxyz, feat.reshape(B, S, 256)


# speedup vs baseline: 12.1825x; 1.6110x over previous
import functools

import jax
import jax.numpy as jnp
from jax import lax
from jax.experimental import pallas as pl
from jax.experimental.pallas import tpu as pltpu
from jax.experimental.pallas import tpu_sc as plsc

B = 8
N = 4096
S = 512
K = 32
EPS = 1e-5


def _fps_body(px_ref, py_ref, pz_ref, cx_ref, cy_ref, cz_ref):
    px = px_ref[...]
    py = py_ref[...]
    pz = pz_ref[...]
    lane_n = lax.broadcasted_iota(jnp.int32, (B, N), 1)
    lane_s = lax.broadcasted_iota(jnp.int32, (B, S), 1)
    b2 = (px * px + py * py) + pz * pz

    def r16(v):
        return v.astype(jnp.bfloat16).astype(jnp.float32)

    px16 = r16(px)
    py16 = r16(py)
    pz16 = r16(pz)

    cx0 = px[:, 0:1]
    cy0 = py[:, 0:1]
    cz0 = pz[:, 0:1]
    a2 = (cx0 * cx0 + cy0 * cy0) + cz0 * cz0
    cross = (px16 * r16(cx0) + py16 * r16(cy0)) + pz16 * r16(cz0)
    d = (a2 + b2) - 2.0 * cross

    zero_s = jnp.zeros((B, S), jnp.float32)
    sel0 = lane_s == 0
    accx = jnp.where(sel0, cx0, zero_s)
    accy = jnp.where(sel0, cy0, zero_s)
    accz = jnp.where(sel0, cz0, zero_s)

    def body(c, carry):
        d, accx, accy, accz = carry
        m = jnp.max(d, axis=1, keepdims=True)
        ni = jnp.min(jnp.where(d == m, lane_n, N), axis=1, keepdims=True)
        onehot = (lane_n == ni).astype(jnp.float32)
        nx = jnp.sum(px * onehot, axis=1, keepdims=True)
        ny = jnp.sum(py * onehot, axis=1, keepdims=True)
        nz = jnp.sum(pz * onehot, axis=1, keepdims=True)
        selc = lane_s == c
        accx = jnp.where(selc, nx, accx)
        accy = jnp.where(selc, ny, accy)
        accz = jnp.where(selc, nz, accz)
        a2n = (nx * nx + ny * ny) + nz * nz
        crossn = (px16 * r16(nx) + py16 * r16(ny)) + pz16 * r16(nz)
        nd = (a2n + b2) - 2.0 * crossn
        d = jnp.minimum(d, nd)
        return d, accx, accy, accz

    _, accx, accy, accz = lax.fori_loop(1, S, body, (d, accx, accy, accz))
    cx_ref[...] = accx
    cy_ref[...] = accy
    cz_ref[...] = accz


def _fps(px, py, pz, interpret=False):
    out = jax.ShapeDtypeStruct((B, S), jnp.float32)
    return pl.pallas_call(
        _fps_body,
        out_shape=(out, out, out),
        interpret=interpret,
    )(px, py, pz)


_TS = 512


def _knn_body(p_ref, c_ref, kidx_ref):
    P = p_ref[0]
    C = c_ref[0]
    px = P[:, 0:1]
    py = P[:, 1:2]
    pz = P[:, 2:3]
    b2 = (px * px + py * py) + pz * pz
    cx = C[0:1, :]
    cy = C[1:2, :]
    cz = C[2:3, :]
    a2 = (cx * cx + cy * cy) + cz * cz
    cross = lax.dot_general(P, C, (((1,), (0,)), ((), ())),
                            preferred_element_type=jnp.float32)
    d = (a2 + b2) - 2.0 * cross

    iota0 = lax.broadcasted_iota(jnp.int32, (N, _TS), 0)

    def rnd(k, d):
        m = jnp.min(d, axis=0, keepdims=True)
        ni = jnp.min(jnp.where(d == m, iota0, N), axis=0,
                     keepdims=True)
        kidx_ref[0, 0, pl.ds(k, 1), :] = ni
        return jnp.where(iota0 == ni, jnp.inf, d)

    for k in range(K):
        d = rnd(k, d)


def _knn(points, cT, interpret=False):
    grid = (B, S // _TS)
    return pl.pallas_call(
        _knn_body,
        grid=grid,
        in_specs=[
            pl.BlockSpec((1, N, 3), lambda b, t: (b, 0, 0)),
            pl.BlockSpec((1, 3, _TS), lambda b, t: (b, 0, t)),
        ],
        out_specs=pl.BlockSpec((1, 1, K, _TS), lambda b, t: (b, t, 0, 0)),
        out_shape=jax.ShapeDtypeStruct((B, S // _TS, K, _TS), jnp.int32),
        compiler_params=pltpu.CompilerParams(
            dimension_semantics=("parallel", "parallel")),
        interpret=interpret,
    )(points, cT)


_M = B * S * K
_CH = 128


def _sc_gather(table, idx):
    info = plsc.get_sparse_core_info()
    nw = info.num_cores * info.num_subcores
    rows_per_w = _M // nw
    n_ch = rows_per_w // _CH
    mesh = plsc.VectorSubcoreMesh(core_axis_name="c", subcore_axis_name="s")

    @functools.partial(
        pl.kernel,
        mesh=mesh,
        out_type=jax.ShapeDtypeStruct((_M, 128), jnp.float32),
        scratch_types=[
            pltpu.VMEM((_CH,), jnp.int32),
            pltpu.VMEM((_CH, 128), jnp.float32),
            pltpu.SemaphoreType.DMA,
        ],
    )
    def k(table_hbm, idx_hbm, out_hbm, idx_v, rows_v, sem):
        wid = lax.axis_index("s") * info.num_cores + lax.axis_index("c")
        base = wid * rows_per_w

        def body(i, _):
            off = base + i * _CH
            pltpu.sync_copy(idx_hbm.at[pl.ds(off, _CH)], idx_v)
            pltpu.async_copy(table_hbm.at[idx_v], rows_v, sem).wait()
            pltpu.sync_copy(rows_v, out_hbm.at[pl.ds(off, _CH)])
            return 0

        lax.fori_loop(0, n_ch, body, 0)

    return k(table, idx)


_TM = 2048
_G = _TM // K


def _rowsum8(z):
    return jnp.sum(z.reshape(_TM // 8, 8, z.shape[1]), axis=0)


def _sub_center(z, t):
    c = z.shape[1]
    z3 = z.reshape(_G, K, c)
    return (z3 - t[:, None, :]).reshape(_TM, c)


def _p1_body(x_ref, cp_ref, w1_ref, s_ref, q_ref):
    @pl.when(pl.program_id(0) == 0)
    def _init():
        s_ref[...] = jnp.zeros_like(s_ref)
        q_ref[...] = jnp.zeros_like(q_ref)

    x = x_ref[...][:, :64]
    cp = cp_ref[...]
    w1 = w1_ref[...]
    z = lax.dot_general(x, w1, (((1,), (1,)), ((), ())),
                        preferred_element_type=jnp.float32)
    t = lax.dot_general(cp, w1, (((1,), (1,)), ((), ())),
                        preferred_element_type=jnp.float32)
    z = _sub_center(z, t)
    s_ref[...] += _rowsum8(z)
    q_ref[...] += _rowsum8(z * z)


def _layer1(x, cp, w1f, b1f):
    z = lax.dot_general(x, w1f, (((1,), (1,)), ((), ())),
                        preferred_element_type=jnp.float32)
    t = lax.dot_general(cp, w1f, (((1,), (1,)), ((), ())),
                        preferred_element_type=jnp.float32)
    return jnp.maximum(_sub_center(z, t) + b1f, 0.0)


def _p2_body(x_ref, cp_ref, w1f_ref, b1f_ref, w2_ref, s_ref, q_ref):
    @pl.when(pl.program_id(0) == 0)
    def _init():
        s_ref[...] = jnp.zeros_like(s_ref)
        q_ref[...] = jnp.zeros_like(q_ref)

    a1 = _layer1(x_ref[...][:, :64], cp_ref[...], w1f_ref[...], b1f_ref[...])
    z2 = lax.dot_general(a1, w2_ref[...], (((1,), (1,)), ((), ())),
                         preferred_element_type=jnp.float32)
    s_ref[...] += _rowsum8(z2)
    q_ref[...] += _rowsum8(z2 * z2)


def _p3_body(x_ref, cp_ref, w1f_ref, b1f_ref, w2f_ref, b2f_ref,
             sm_ref, m_ref):
    @pl.when(pl.program_id(0) == 0)
    def _init():
        sm_ref[...] = jnp.zeros_like(sm_ref)
        m_ref[...] = jnp.zeros_like(m_ref)

    a1 = _layer1(x_ref[...][:, :64], cp_ref[...], w1f_ref[...], b1f_ref[...])
    z2 = lax.dot_general(a1, w2f_ref[...], (((1,), (1,)), ((), ())),
                         preferred_element_type=jnp.float32)
    a2 = jnp.maximum(z2 + b2f_ref[...], 0.0)
    sm_ref[...] += lax.dot_general(a2, a2, (((0,), (0,)), ((), ())),
                                   preferred_element_type=jnp.float32)
    m_ref[...] += _rowsum8(a2)


def _p4_body(x_ref, cp_ref, w1f_ref, b1f_ref, w2f_ref, b2f_ref,
             w3f_ref, b3f_ref, o_ref):
    a1 = _layer1(x_ref[...][:, :64], cp_ref[...], w1f_ref[...], b1f_ref[...])
    z2 = lax.dot_general(a1, w2f_ref[...], (((1,), (1,)), ((), ())),
                         preferred_element_type=jnp.float32)
    a2 = jnp.maximum(z2 + b2f_ref[...], 0.0)
    z3 = lax.dot_general(a2, w3f_ref[...], (((1,), (1,)), ((), ())),
                         preferred_element_type=jnp.float32)
    a3 = jnp.maximum(z3 + b3f_ref[...], 0.0)
    o_ref[...] = jnp.max(a3.reshape(_G, K, 256), axis=1)


def _mlp(x0, cpad, W1, g1, b1, W2, g2, b2, W3, g3, b3, interpret=False):
    grid = (_M // _TM,)
    row_spec = lambda c: pl.BlockSpec((_TM, c), lambda i: (i, 0))
    x_spec = pl.BlockSpec((_TM, 128), lambda i: (i, 0))
    cp_spec = pl.BlockSpec((_G, 64), lambda i: (i, 0))
    full = lambda a, b: pl.BlockSpec((a, b), lambda i: (0, 0))

    s1, q1 = pl.pallas_call(
        _p1_body, grid=grid,
        in_specs=[x_spec, cp_spec, full(64, 64)],
        out_specs=(full(8, 64), full(8, 64)),
        out_shape=(jax.ShapeDtypeStruct((8, 64), jnp.float32),) * 2,
        interpret=interpret,
    )(x0, cpad, W1)
    mu1 = jnp.sum(s1, axis=0) / _M
    var1 = jnp.sum(q1, axis=0) / _M - mu1 * mu1
    sc1 = g1 * lax.rsqrt(var1 + EPS)
    w1f = W1 * sc1[:, None]
    b1f = (b1 - sc1 * mu1)[None, :]

    s2, q2 = pl.pallas_call(
        _p2_body, grid=grid,
        in_specs=[x_spec, cp_spec, full(64, 64), full(1, 64),
                  full(128, 64)],
        out_specs=(full(8, 128), full(8, 128)),
        out_shape=(jax.ShapeDtypeStruct((8, 128), jnp.float32),) * 2,
        interpret=interpret,
    )(x0, cpad, w1f, b1f, W2)
    mu2 = jnp.sum(s2, axis=0) / _M
    var2 = jnp.sum(q2, axis=0) / _M - mu2 * mu2
    sc2 = g2 * lax.rsqrt(var2 + EPS)
    w2f = W2 * sc2[:, None]
    b2f = (b2 - sc2 * mu2)[None, :]

    sm, m2 = pl.pallas_call(
        _p3_body, grid=grid,
        in_specs=[x_spec, cp_spec, full(64, 64), full(1, 64),
                  full(128, 64), full(1, 128)],
        out_specs=(full(128, 128), full(8, 128)),
        out_shape=(jax.ShapeDtypeStruct((128, 128), jnp.float32),
                   jax.ShapeDtypeStruct((8, 128), jnp.float32)),
        interpret=interpret,
    )(x0, cpad, w1f, b1f, w2f, b2f)
    mu_a2 = jnp.sum(m2, axis=0) / _M
    mu3 = W3 @ mu_a2
    e3 = jnp.sum((W3 @ (sm / _M)) * W3, axis=1)
    var3 = e3 - mu3 * mu3
    sc3 = g3 * lax.rsqrt(var3 + EPS)
    w3f = W3 * sc3[:, None]
    b3f = (b3 - sc3 * mu3)[None, :]

    out = pl.pallas_call(
        _p4_body, grid=grid,
        in_specs=[x_spec, cp_spec, full(64, 64), full(1, 64),
                  full(128, 64), full(1, 128), full(256, 128), full(1, 256)],
        out_specs=pl.BlockSpec((_G, 256), lambda i: (i, 0)),
        out_shape=jax.ShapeDtypeStruct((_M // K, 256), jnp.float32),
        compiler_params=pltpu.CompilerParams(
            dimension_semantics=("parallel",)),
        interpret=interpret,
    )(x0, cpad, w1f, b1f, w2f, b2f, w3f, b3f)
    return out


def kernel(points, features, W1, g1, b1, W2, g2, b2, W3, g3, b3):
    px = points[:, :, 0]
    py = points[:, :, 1]
    pz = points[:, :, 2]
    cx, cy, cz = _fps(px, py, pz)
    cxyz = jnp.stack([cx, cy, cz], axis=-1)
    cT = jnp.stack([cx, cy, cz], axis=1)

    kidx4 = _knn(points, cT)
    kidx = jnp.transpose(kidx4, (0, 1, 3, 2)).reshape(B, S, K)

    flat_idx = (kidx + (jnp.arange(B, dtype=jnp.int32) * N)[:, None, None])
    flat_idx = flat_idx.reshape(_M)
    table = jnp.concatenate([points, features], axis=-1).reshape(B * N, 64)
    table = jnp.pad(table, ((0, 0), (0, 64)))
    x0 = _sc_gather(table, flat_idx)

    cpad = jnp.pad(cxyz.reshape(B * S, 3), ((0, 0), (0, 61)))
    feat = _mlp(x0, cpad, W1, g1, b1, W2, g2, b2, W3, g3, b3)
    return cxyz, feat.reshape(B, S, 256)


# speedup vs baseline: 12.9097x; 1.0597x over previous
import functools

import jax
import jax.numpy as jnp
from jax import lax
from jax.experimental import pallas as pl
from jax.experimental.pallas import tpu as pltpu
from jax.experimental.pallas import tpu_sc as plsc

B = 8
N = 4096
S = 512
K = 32
EPS = 1e-5


def _fps_body(px_ref, py_ref, pz_ref, cx_ref, cy_ref, cz_ref):
    px = px_ref[...]
    py = py_ref[...]
    pz = pz_ref[...]
    lane_n = lax.broadcasted_iota(jnp.int32, (B, N), 1)
    lane_s = lax.broadcasted_iota(jnp.int32, (B, S), 1)
    b2 = (px * px + py * py) + pz * pz

    def r16(v):
        return v.astype(jnp.bfloat16).astype(jnp.float32)

    px16 = r16(px)
    py16 = r16(py)
    pz16 = r16(pz)

    cx0 = px[:, 0:1]
    cy0 = py[:, 0:1]
    cz0 = pz[:, 0:1]
    a2 = (cx0 * cx0 + cy0 * cy0) + cz0 * cz0
    cross = (px16 * r16(cx0) + py16 * r16(cy0)) + pz16 * r16(cz0)
    d = (a2 + b2) - 2.0 * cross

    zero_s = jnp.zeros((B, S), jnp.float32)
    sel0 = lane_s == 0
    accx = jnp.where(sel0, cx0, zero_s)
    accy = jnp.where(sel0, cy0, zero_s)
    accz = jnp.where(sel0, cz0, zero_s)

    def body(c, carry):
        d, accx, accy, accz = carry
        m = jnp.max(d, axis=1, keepdims=True)
        ni = jnp.min(jnp.where(d == m, lane_n, N), axis=1, keepdims=True)
        onehot = (lane_n == ni).astype(jnp.float32)
        nx = jnp.sum(px * onehot, axis=1, keepdims=True)
        ny = jnp.sum(py * onehot, axis=1, keepdims=True)
        nz = jnp.sum(pz * onehot, axis=1, keepdims=True)
        selc = lane_s == c
        accx = jnp.where(selc, nx, accx)
        accy = jnp.where(selc, ny, accy)
        accz = jnp.where(selc, nz, accz)
        a2n = (nx * nx + ny * ny) + nz * nz
        crossn = (px16 * r16(nx) + py16 * r16(ny)) + pz16 * r16(nz)
        nd = (a2n + b2) - 2.0 * crossn
        d = jnp.minimum(d, nd)
        return d, accx, accy, accz

    _, accx, accy, accz = lax.fori_loop(1, S, body, (d, accx, accy, accz))
    cx_ref[...] = accx
    cy_ref[...] = accy
    cz_ref[...] = accz


def _fps(px, py, pz, interpret=False):
    out = jax.ShapeDtypeStruct((B, S), jnp.float32)
    return pl.pallas_call(
        _fps_body,
        out_shape=(out, out, out),
        interpret=interpret,
    )(px, py, pz)


_TS = 512


def _knn_body(p_ref, c_ref, kidx_ref):
    P = p_ref[0]
    C = c_ref[0]
    px = P[:, 0:1]
    py = P[:, 1:2]
    pz = P[:, 2:3]
    b2 = (px * px + py * py) + pz * pz
    cx = C[0:1, :]
    cy = C[1:2, :]
    cz = C[2:3, :]
    a2 = (cx * cx + cy * cy) + cz * cz
    cross = lax.dot_general(P, C, (((1,), (0,)), ((), ())),
                            preferred_element_type=jnp.float32)
    d = (a2 + b2) - 2.0 * cross

    iota0 = lax.broadcasted_iota(jnp.int32, (N, _TS), 0)

    def rnd(k, d):
        m = jnp.min(d, axis=0, keepdims=True)
        ni = jnp.min(jnp.where(d == m, iota0, N), axis=0,
                     keepdims=True)
        kidx_ref[0, 0, pl.ds(k, 1), :] = ni
        return jnp.where(iota0 == ni, jnp.inf, d)

    for k in range(K):
        d = rnd(k, d)


def _knn(points, cT, interpret=False):
    grid = (B, S // _TS)
    return pl.pallas_call(
        _knn_body,
        grid=grid,
        in_specs=[
            pl.BlockSpec((1, N, 3), lambda b, t: (b, 0, 0)),
            pl.BlockSpec((1, 3, _TS), lambda b, t: (b, 0, t)),
        ],
        out_specs=pl.BlockSpec((1, 1, K, _TS), lambda b, t: (b, t, 0, 0)),
        out_shape=jax.ShapeDtypeStruct((B, S // _TS, K, _TS), jnp.int32),
        compiler_params=pltpu.CompilerParams(
            dimension_semantics=("parallel", "parallel")),
        interpret=interpret,
    )(points, cT)


_M = B * S * K
_CH = 128


def _sc_gather(table, idx):
    info = plsc.get_sparse_core_info()
    nw = info.num_cores * info.num_subcores
    rows_per_w = _M // nw
    n_ch = rows_per_w // _CH
    mesh = plsc.VectorSubcoreMesh(core_axis_name="c", subcore_axis_name="s")

    @functools.partial(
        pl.kernel,
        mesh=mesh,
        out_type=jax.ShapeDtypeStruct((_M, 128), jnp.float32),
        scratch_types=[
            pltpu.VMEM((_CH,), jnp.int32),
            pltpu.VMEM((_CH, 128), jnp.float32),
            pltpu.SemaphoreType.DMA,
        ],
    )
    def k(table_hbm, idx_hbm, out_hbm, idx_v, rows_v, sem):
        wid = lax.axis_index("s") * info.num_cores + lax.axis_index("c")
        base = wid * rows_per_w

        def body(i, _):
            off = base + i * _CH
            pltpu.sync_copy(idx_hbm.at[pl.ds(off, _CH)], idx_v)
            pltpu.async_copy(table_hbm.at[idx_v], rows_v, sem).wait()
            pltpu.sync_copy(rows_v, out_hbm.at[pl.ds(off, _CH)])
            return 0

        lax.fori_loop(0, n_ch, body, 0)

    return k(table, idx)


_TM = 4096
_G = _TM // K


def _rowsum8(z):
    return jnp.sum(z.reshape(_TM // 8, 8, z.shape[1]), axis=0)


def _sub_center(z, t):
    c = z.shape[1]
    z3 = z.reshape(_G, K, c)
    return (z3 - t[:, None, :]).reshape(_TM, c)


def _p1_body(x_ref, cp_ref, w1_ref, s_ref, q_ref):
    @pl.when(pl.program_id(0) == 0)
    def _init():
        s_ref[...] = jnp.zeros_like(s_ref)
        q_ref[...] = jnp.zeros_like(q_ref)

    x = x_ref[...][:, :64]
    cp = cp_ref[...]
    w1 = w1_ref[...]
    z = lax.dot_general(x, w1, (((1,), (1,)), ((), ())),
                        preferred_element_type=jnp.float32)
    t = lax.dot_general(cp, w1, (((1,), (1,)), ((), ())),
                        preferred_element_type=jnp.float32)
    z = _sub_center(z, t)
    s_ref[...] += _rowsum8(z)
    q_ref[...] += _rowsum8(z * z)


def _layer1(x, cp, w1f, b1f):
    z = lax.dot_general(x, w1f, (((1,), (1,)), ((), ())),
                        preferred_element_type=jnp.float32)
    t = lax.dot_general(cp, w1f, (((1,), (1,)), ((), ())),
                        preferred_element_type=jnp.float32)
    return jnp.maximum(_sub_center(z, t) + b1f, 0.0)


def _p2_body(x_ref, cp_ref, w1f_ref, b1f_ref, w2_ref, s_ref, q_ref):
    @pl.when(pl.program_id(0) == 0)
    def _init():
        s_ref[...] = jnp.zeros_like(s_ref)
        q_ref[...] = jnp.zeros_like(q_ref)

    a1 = _layer1(x_ref[...][:, :64], cp_ref[...], w1f_ref[...], b1f_ref[...])
    z2 = lax.dot_general(a1, w2_ref[...], (((1,), (1,)), ((), ())),
                         preferred_element_type=jnp.float32)
    s_ref[...] += _rowsum8(z2)
    q_ref[...] += _rowsum8(z2 * z2)


def _p3_body(x_ref, cp_ref, w1f_ref, b1f_ref, w2f_ref, b2f_ref,
             sm_ref, m_ref):
    @pl.when(pl.program_id(0) == 0)
    def _init():
        sm_ref[...] = jnp.zeros_like(sm_ref)
        m_ref[...] = jnp.zeros_like(m_ref)

    a1 = _layer1(x_ref[...][:, :64], cp_ref[...], w1f_ref[...], b1f_ref[...])
    z2 = lax.dot_general(a1, w2f_ref[...], (((1,), (1,)), ((), ())),
                         preferred_element_type=jnp.float32)
    a2 = jnp.maximum(z2 + b2f_ref[...], 0.0)
    sm_ref[...] += lax.dot_general(a2, a2, (((0,), (0,)), ((), ())),
                                   preferred_element_type=jnp.float32)
    m_ref[...] += _rowsum8(a2)


def _p4_body(x_ref, cp_ref, w1f_ref, b1f_ref, w2f_ref, b2f_ref,
             w3f_ref, b3f_ref, o_ref):
    a1 = _layer1(x_ref[...][:, :64], cp_ref[...], w1f_ref[...], b1f_ref[...])
    z2 = lax.dot_general(a1, w2f_ref[...], (((1,), (1,)), ((), ())),
                         preferred_element_type=jnp.float32)
    a2 = jnp.maximum(z2 + b2f_ref[...], 0.0)
    z3 = lax.dot_general(a2, w3f_ref[...], (((1,), (1,)), ((), ())),
                         preferred_element_type=jnp.float32)
    a3 = jnp.maximum(z3 + b3f_ref[...], 0.0)
    o_ref[...] = jnp.max(a3.reshape(_G, K, 256), axis=1)


def _mlp(x0, cpad, W1, g1, b1, W2, g2, b2, W3, g3, b3, interpret=False):
    grid = (_M // _TM,)
    row_spec = lambda c: pl.BlockSpec((_TM, c), lambda i: (i, 0))
    x_spec = pl.BlockSpec((_TM, 128), lambda i: (i, 0))
    cp_spec = pl.BlockSpec((_G, 64), lambda i: (i, 0))
    full = lambda a, b: pl.BlockSpec((a, b), lambda i: (0, 0))

    s1, q1 = pl.pallas_call(
        _p1_body, grid=grid,
        in_specs=[x_spec, cp_spec, full(64, 64)],
        out_specs=(full(8, 64), full(8, 64)),
        out_shape=(jax.ShapeDtypeStruct((8, 64), jnp.float32),) * 2,
        interpret=interpret,
    )(x0, cpad, W1)
    mu1 = jnp.sum(s1, axis=0) / _M
    var1 = jnp.sum(q1, axis=0) / _M - mu1 * mu1
    sc1 = g1 * lax.rsqrt(var1 + EPS)
    w1f = W1 * sc1[:, None]
    b1f = (b1 - sc1 * mu1)[None, :]

    s2, q2 = pl.pallas_call(
        _p2_body, grid=grid,
        in_specs=[x_spec, cp_spec, full(64, 64), full(1, 64),
                  full(128, 64)],
        out_specs=(full(8, 128), full(8, 128)),
        out_shape=(jax.ShapeDtypeStruct((8, 128), jnp.float32),) * 2,
        interpret=interpret,
    )(x0, cpad, w1f, b1f, W2)
    mu2 = jnp.sum(s2, axis=0) / _M
    var2 = jnp.sum(q2, axis=0) / _M - mu2 * mu2
    sc2 = g2 * lax.rsqrt(var2 + EPS)
    w2f = W2 * sc2[:, None]
    b2f = (b2 - sc2 * mu2)[None, :]

    sm, m2 = pl.pallas_call(
        _p3_body, grid=grid,
        in_specs=[x_spec, cp_spec, full(64, 64), full(1, 64),
                  full(128, 64), full(1, 128)],
        out_specs=(full(128, 128), full(8, 128)),
        out_shape=(jax.ShapeDtypeStruct((128, 128), jnp.float32),
                   jax.ShapeDtypeStruct((8, 128), jnp.float32)),
        interpret=interpret,
    )(x0, cpad, w1f, b1f, w2f, b2f)
    mu_a2 = jnp.sum(m2, axis=0) / _M
    mu3 = W3 @ mu_a2
    e3 = jnp.sum((W3 @ (sm / _M)) * W3, axis=1)
    var3 = e3 - mu3 * mu3
    sc3 = g3 * lax.rsqrt(var3 + EPS)
    w3f = W3 * sc3[:, None]
    b3f = (b3 - sc3 * mu3)[None, :]

    out = pl.pallas_call(
        _p4_body, grid=grid,
        in_specs=[x_spec, cp_spec, full(64, 64), full(1, 64),
                  full(128, 64), full(1, 128), full(256, 128), full(1, 256)],
        out_specs=pl.BlockSpec((_G, 256), lambda i: (i, 0)),
        out_shape=jax.ShapeDtypeStruct((_M // K, 256), jnp.float32),
        compiler_params=pltpu.CompilerParams(
            dimension_semantics=("parallel",)),
        interpret=interpret,
    )(x0, cpad, w1f, b1f, w2f, b2f, w3f, b3f)
    return out


def kernel(points, features, W1, g1, b1, W2, g2, b2, W3, g3, b3):
    px = points[:, :, 0]
    py = points[:, :, 1]
    pz = points[:, :, 2]
    cx, cy, cz = _fps(px, py, pz)
    cxyz = jnp.stack([cx, cy, cz], axis=-1)
    cT = jnp.stack([cx, cy, cz], axis=1)

    kidx4 = _knn(points, cT)
    kidx = jnp.transpose(kidx4, (0, 1, 3, 2)).reshape(B, S, K)

    flat_idx = (kidx + (jnp.arange(B, dtype=jnp.int32) * N)[:, None, None])
    flat_idx = flat_idx.reshape(_M)
    table = jnp.concatenate([points, features], axis=-1).reshape(B * N, 64)
    table = jnp.pad(table, ((0, 0), (0, 64)))
    x0 = _sc_gather(table, flat_idx)

    cpad = jnp.pad(cxyz.reshape(B * S, 3), ((0, 0), (0, 61)))
    feat = _mlp(x0, cpad, W1, g1, b1, W2, g2, b2, W3, g3, b3)
    return cxyz, feat.reshape(B, S, 256)
